# baseline (speedup 1.0000x reference)
import functools

import jax
import jax.numpy as jnp
from jax import lax
from jax.experimental import pallas as pl
from jax.experimental.pallas import tpu as pltpu
from jax.experimental.pallas import tpu_sc as plsc

_NS = 16
_CHUNK = 64


def _sc_deg(dst_p, NP):
    EP = dst_p.shape[0]
    NW = 2 * _NS
    e_per = EP // NW
    full, rem = divmod(e_per, 2048)
    mesh = plsc.VectorSubcoreMesh(core_axis_name="c", subcore_axis_name="s")

    @functools.partial(
        pl.kernel,
        out_type=jax.ShapeDtypeStruct((NW, NP), jnp.float32),
        mesh=mesh,
        scratch_types=[
            pltpu.VMEM((2048,), jnp.int32),
            pltpu.VMEM((NP,), jnp.float32),
        ],
        compiler_params=pltpu.CompilerParams(needs_layout_passes=False),
    )
    def k(dst_h, out_h, dbuf, hist):
        c = lax.axis_index("c")
        s = lax.axis_index("s")
        w = s * 2 + c

        def zi(i, _):
            hist[pl.ds(i * 16, 16)] = jnp.zeros((16,), jnp.float32)
            return 0

        lax.fori_loop(0, NP // 16, zi, 0)
        ones = jnp.ones((16,), jnp.float32)
        base = w * e_per

        def scat(i, _):
            idx = dbuf[pl.ds(i * 16, 16)]
            plsc.addupdate_scatter(hist, [idx], ones)
            return 0

        def do_chunk(off, n):
            pltpu.sync_copy(dst_h.at[pl.ds(base + off, n)], dbuf.at[pl.ds(0, n)])
            lax.fori_loop(0, n // 16, scat, 0)

        for j in range(full):
            do_chunk(j * 2048, 2048)
        if rem:
            do_chunk(full * 2048, rem)
        pltpu.sync_copy(hist, out_h.at[w])

    return k(dst_p)


_BLK = 8
_RING = 4


_ZR = 32


def _zero_acc(zbuf, acc, s, rows_per, D):
    def zrow(r, _):
        def zcol(j, _):
            zbuf[r, pl.ds(j * 16, 16)] = jnp.zeros((16,), jnp.float32)
            return 0
        lax.fori_loop(0, D // 16, zcol, 0)
        return 0

    lax.fori_loop(0, _ZR, zrow, 0)

    def zcp(i, _):
        pltpu.sync_copy(zbuf, acc.at[pl.ds(s * rows_per + i * _ZR, _ZR)])
        return 0

    lax.fori_loop(0, rows_per // _ZR, zcp, 0)
    plsc.subcore_barrier()


def _agg_pipeline(src_h, dst_h, hs_h, acc, sbuf, dbuf, rows, sems,
                  base_row, nblk):
    def fire(pb, j, r):
        pltpu.async_copy(hs_h.at[sbuf.at[pb, j]], rows.at[r], sems[r])

    def drain(r):
        pltpu.make_async_copy(hs_h.at[sbuf.at[0, 0]], rows.at[r], sems[r]).wait()

    pltpu.sync_copy(src_h.at[pl.ds(base_row, _BLK)], sbuf.at[0])
    pltpu.sync_copy(dst_h.at[pl.ds(base_row, _BLK)], dbuf.at[0])
    for j in range(_RING - 1):
        fire(0, j, j)

    def block(b, _):
        par = lax.rem(b, 2)
        nxt = 1 - par

        @pl.when(b + 1 < nblk)
        def _():
            r1 = base_row + (b + 1) * _BLK
            pltpu.sync_copy(src_h.at[pl.ds(r1, _BLK)], sbuf.at[nxt])
            pltpu.sync_copy(dst_h.at[pl.ds(r1, _BLK)], dbuf.at[nxt])

        for j in range(_BLK):
            r = j % _RING
            rn = (j + _RING - 1) % _RING
            drain(r)
            jn = j + _RING - 1
            if jn < _BLK:
                fire(par, jn, rn)
            else:
                @pl.when(b + 1 < nblk)
                def _():
                    fire(nxt, jn - _BLK, rn)
            pltpu.sync_copy(rows.at[r], acc.at[dbuf.at[par, j]], add=True)
        return 0

    lax.fori_loop(0, nblk, block, 0)


def _agg_scratch(NP, D):
    return [
        pltpu.VMEM((2, _BLK, _CHUNK), jnp.int32),
        pltpu.VMEM((2, _BLK, _CHUNK), jnp.int32),
        pltpu.VMEM((_RING, _CHUNK, D), jnp.float32),
        pltpu.VMEM((_ZR, D), jnp.float32),
        pltpu.VMEM_SHARED((NP, D), jnp.float32),
    ] + [pltpu.SemaphoreType.DMA] * _RING


def _sc_agg(src_p, dst_p, hs0, hs1, NP, Dh):
    NR = src_p.shape[0]
    rows_w = NR // _NS
    nblk = rows_w // _BLK
    rows_per = NP // _NS
    mesh = plsc.VectorSubcoreMesh(core_axis_name="c", subcore_axis_name="s")

    @functools.partial(
        pl.kernel,
        out_type=(
            jax.ShapeDtypeStruct((NP, Dh), jnp.float32),
            jax.ShapeDtypeStruct((NP, Dh), jnp.float32),
        ),
        mesh=mesh,
        scratch_types=_agg_scratch(NP, Dh),
    )
    def k(src_h, dst_h, hs0_h, hs1_h, out0_h, out1_h,
          sbuf, dbuf, rows, zbuf, acc, *sems):
        c = lax.axis_index("c")
        s = lax.axis_index("s")
        _zero_acc(zbuf, acc, s, rows_per, Dh)
        base_row = s * rows_w

        def run(hs_h, out_h):
            _agg_pipeline(src_h, dst_h, hs_h, acc, sbuf, dbuf, rows,
                          sems, base_row, nblk)
            plsc.subcore_barrier()
            pltpu.sync_copy(acc.at[pl.ds(s * rows_per, rows_per)],
                            out_h.at[pl.ds(s * rows_per, rows_per)])

        @pl.when(c == 0)
        def _():
            run(hs0_h, out0_h)

        @pl.when(c == 1)
        def _():
            run(hs1_h, out1_h)

    return k(src_p, dst_p, hs0, hs1)


def _sc_agg_esplit(src_p, dst_p, hs, NP, D, blk0=None):
    NR = src_p.shape[0]
    rows_w = NR // (2 * _NS)
    nblk = rows_w // _BLK
    nblk0 = nblk if blk0 is None else blk0
    nblk1 = 2 * nblk - nblk0
    rows_per = NP // _NS
    mesh = plsc.VectorSubcoreMesh(core_axis_name="c", subcore_axis_name="s")

    @functools.partial(
        pl.kernel,
        out_type=(
            jax.ShapeDtypeStruct((NP, D), jnp.float32),
            jax.ShapeDtypeStruct((NP, D), jnp.float32),
        ),
        mesh=mesh,
        scratch_types=_agg_scratch(NP, D),
    )
    def k(src_h, dst_h, hs_h, out0_h, out1_h,
          sbuf, dbuf, rows, zbuf, acc, *sems):
        c = lax.axis_index("c")
        s = lax.axis_index("s")
        _zero_acc(zbuf, acc, s, rows_per, D)
        rw0 = nblk0 * _BLK
        rw1 = nblk1 * _BLK
        base_row = jnp.where(c == 0, s * rw0, _NS * rw0 + s * rw1)
        my_nblk = jnp.where(c == 0, nblk0, nblk1)

        @pl.when(my_nblk > 0)
        def _():
            _agg_pipeline(src_h, dst_h, hs_h, acc, sbuf, dbuf, rows,
                          sems, base_row, my_nblk)

        plsc.subcore_barrier()

        def out(out_h):
            pltpu.sync_copy(acc.at[pl.ds(s * rows_per, rows_per)],
                            out_h.at[pl.ds(s * rows_per, rows_per)])

        @pl.when(c == 0)
        def _():
            out(out0_h)

        @pl.when(c == 1)
        def _():
            out(out1_h)

    return k(src_p, dst_p, hs)


def _tc_pre(xp, W1, hist, bm=1024):
    NP, Din = xp.shape
    Dh = W1.shape[1] // 2
    NW = hist.shape[0]

    def body(x_ref, w_ref, h_ref, hs0_ref, hs1_ref, dis_ref):
        deg = jnp.sum(h_ref[...], axis=0) + 1.0
        dis = lax.rsqrt(deg)
        xs = x_ref[...] * dis[:, None]
        hs = jnp.dot(xs, w_ref[...], preferred_element_type=jnp.float32)
        hs0_ref[...] = hs[:, :Dh]
        hs1_ref[...] = hs[:, Dh:]
        dis_ref[...] = dis[:, None]

    return pl.pallas_call(
        body,
        grid=(NP // bm,),
        in_specs=[
            pl.BlockSpec((bm, Din), lambda i: (i, 0)),
            pl.BlockSpec((Din, 2 * Dh), lambda i: (0, 0)),
            pl.BlockSpec((NW, bm), lambda i: (0, i)),
        ],
        out_specs=[
            pl.BlockSpec((bm, Dh), lambda i: (i, 0)),
            pl.BlockSpec((bm, Dh), lambda i: (i, 0)),
            pl.BlockSpec((bm, 1), lambda i: (i, 0)),
        ],
        out_shape=[
            jax.ShapeDtypeStruct((NP, Dh), jnp.float32),
            jax.ShapeDtypeStruct((NP, Dh), jnp.float32),
            jax.ShapeDtypeStruct((NP, 1), jnp.float32),
        ],
    )(xp, W1, hist)


def _tc_mid(agg0, agg1, hs0, hs1, dis, b1, W2, bm=1024):
    NP, Dh = agg0.shape
    Dout = W2.shape[1]
    Dq = Dout // 2

    def body(a0, a1, h0, h1, d_ref, b_ref, w_ref, o_ref):
        d = d_ref[...]
        b = b_ref[...]
        w = w_ref[...]
        t0 = jnp.maximum(d * (a0[...] + h0[...]) + b[0:1, :Dh], 0.0)
        t1 = jnp.maximum(d * (a1[...] + h1[...]) + b[0:1, Dh:], 0.0)
        o_ref[...] = (jnp.dot(d * t0, w[:Dh], preferred_element_type=jnp.float32)
                      + jnp.dot(d * t1, w[Dh:], preferred_element_type=jnp.float32))

    rb = pl.BlockSpec((bm, Dh), lambda i: (i, 0))
    return pl.pallas_call(
        body,
        grid=(NP // bm,),
        in_specs=[
            rb, rb, rb, rb,
            pl.BlockSpec((bm, 1), lambda i: (i, 0)),
            pl.BlockSpec((8, 2 * Dh), lambda i: (0, 0)),
            pl.BlockSpec((2 * Dh, Dout), lambda i: (0, 0)),
        ],
        out_specs=pl.BlockSpec((bm, Dout), lambda i: (i, 0)),
        out_shape=jax.ShapeDtypeStruct((NP, Dout), jnp.float32),
    )(agg0, agg1, hs0, hs1, dis, b1, W2)


def _tc_post(p0, p1, hs2, dis, b2, bm=1024):
    NP, D = hs2.shape

    def body(a0, a1, h_ref, d_ref, b_ref, o_ref):
        d = d_ref[...]
        b = b_ref[...]
        o_ref[...] = jnp.maximum(
            d * (a0[...] + a1[...] + h_ref[...]) + b[0:1, :], 0.0)

    rb = pl.BlockSpec((bm, D), lambda i: (i, 0))
    return pl.pallas_call(
        body,
        grid=(NP // bm,),
        in_specs=[
            rb, rb, rb,
            pl.BlockSpec((bm, 1), lambda i: (i, 0)),
            pl.BlockSpec((8, D), lambda i: (0, 0)),
        ],
        out_specs=pl.BlockSpec((bm, D), lambda i: (i, 0)),
        out_shape=jax.ShapeDtypeStruct((NP, D), jnp.float32),
    )(p0, p1, hs2, dis, b2)


@jax.jit
def _run(x, edge_index, W1, b1, W2, b2):
    N, Din = x.shape
    E = edge_index.shape[1]
    NP = ((N + 2047) // 2048) * 2048
    blk_edges = 2 * _NS * _BLK * _CHUNK
    EP = ((E + blk_edges - 1) // blk_edges) * blk_edges

    src = edge_index[0].astype(jnp.int32)
    dst = edge_index[1].astype(jnp.int32)
    pad = jnp.full((EP - E,), N, dtype=jnp.int32)
    src_p = jnp.concatenate([src, pad])
    dst_p = jnp.concatenate([dst, pad])
    src_2d = src_p.reshape(EP // _CHUNK, _CHUNK)
    dst_2d = dst_p.reshape(EP // _CHUNK, _CHUNK)
    xp = jnp.concatenate([x, jnp.zeros((NP - N, Din), x.dtype)])
    b1r = jnp.broadcast_to(b1[None, :], (8, b1.shape[0]))
    b2r = jnp.broadcast_to(b2[None, :], (8, b2.shape[0]))

    hist = _sc_deg(dst_p, NP)
    hs0, hs1, dis = _tc_pre(xp, W1, hist)
    agg0, agg1 = _sc_agg(src_2d, dst_2d, hs0, hs1, NP, W1.shape[1] // 2)
    hs2 = _tc_mid(agg0, agg1, hs0, hs1, dis, b1r, W2)
    p0, p1 = _sc_agg_esplit(src_2d, dst_2d, hs2, NP, W2.shape[1], blk0=40)
    y = _tc_post(p0, p1, hs2, dis, b2r)
    return y[:N]


def kernel(x, edge_index, W1, b1, W2, b2):
    return _run(x, edge_index, W1, b1, W2, b2)


# speedup vs baseline: 1.1396x; 1.1396x over previous
import functools

import jax
import jax.numpy as jnp
from jax import lax
from jax.experimental import pallas as pl
from jax.experimental.pallas import tpu as pltpu
from jax.experimental.pallas import tpu_sc as plsc

_NS = 16
_CHUNK = 64


def _sc_deg(dst_p, NP):
    EP = dst_p.shape[0]
    NW = 2 * _NS
    e_per = EP // NW
    full, rem = divmod(e_per, 2048)
    mesh = plsc.VectorSubcoreMesh(core_axis_name="c", subcore_axis_name="s")

    @functools.partial(
        pl.kernel,
        out_type=jax.ShapeDtypeStruct((NW, NP), jnp.float32),
        mesh=mesh,
        scratch_types=[
            pltpu.VMEM((2048,), jnp.int32),
            pltpu.VMEM((NP,), jnp.float32),
        ],
        compiler_params=pltpu.CompilerParams(needs_layout_passes=False),
    )
    def k(dst_h, out_h, dbuf, hist):
        c = lax.axis_index("c")
        s = lax.axis_index("s")
        w = s * 2 + c

        def zi(i, _):
            hist[pl.ds(i * 16, 16)] = jnp.zeros((16,), jnp.float32)
            return 0

        lax.fori_loop(0, NP // 16, zi, 0)
        ones = jnp.ones((16,), jnp.float32)
        base = w * e_per

        def scat(i, _):
            idx = dbuf[pl.ds(i * 16, 16)]
            plsc.addupdate_scatter(hist, [idx], ones)
            return 0

        def do_chunk(off, n):
            pltpu.sync_copy(dst_h.at[pl.ds(base + off, n)], dbuf.at[pl.ds(0, n)])
            lax.fori_loop(0, n // 16, scat, 0)

        for j in range(full):
            do_chunk(j * 2048, 2048)
        if rem:
            do_chunk(full * 2048, rem)
        pltpu.sync_copy(hist, out_h.at[w])

    return k(dst_p)


_BLK = 8
_RING = 4


_ZR = 32


def _zero_acc(zbuf, acc, s, rows_per, D):
    def zrow(r, _):
        def zcol(j, _):
            zbuf[r, pl.ds(j * 16, 16)] = jnp.zeros((16,), jnp.float32)
            return 0
        lax.fori_loop(0, D // 16, zcol, 0)
        return 0

    lax.fori_loop(0, _ZR, zrow, 0)

    def zcp(i, _):
        pltpu.sync_copy(zbuf, acc.at[pl.ds(s * rows_per + i * _ZR, _ZR)])
        return 0

    lax.fori_loop(0, rows_per // _ZR, zcp, 0)
    plsc.subcore_barrier()


def _agg_pipeline(src_h, dst_h, hs_h, acc, sbuf, dbuf, rows, sems,
                  base_row, nblk):
    def fire(pb, j, r):
        pltpu.async_copy(hs_h.at[sbuf.at[pb, j]], rows.at[r], sems[r])

    def drain(r):
        pltpu.make_async_copy(hs_h.at[sbuf.at[0, 0]], rows.at[r], sems[r]).wait()

    pltpu.sync_copy(src_h.at[pl.ds(base_row, _BLK)], sbuf.at[0])
    pltpu.sync_copy(dst_h.at[pl.ds(base_row, _BLK)], dbuf.at[0])
    for j in range(_RING - 1):
        fire(0, j, j)

    def block(b, _):
        par = lax.rem(b, 2)
        nxt = 1 - par

        @pl.when(b + 1 < nblk)
        def _():
            r1 = base_row + (b + 1) * _BLK
            pltpu.sync_copy(src_h.at[pl.ds(r1, _BLK)], sbuf.at[nxt])
            pltpu.sync_copy(dst_h.at[pl.ds(r1, _BLK)], dbuf.at[nxt])

        for j in range(_BLK):
            r = j % _RING
            rn = (j + _RING - 1) % _RING
            drain(r)
            jn = j + _RING - 1
            if jn < _BLK:
                fire(par, jn, rn)
            else:
                @pl.when(b + 1 < nblk)
                def _():
                    fire(nxt, jn - _BLK, rn)
            pltpu.sync_copy(rows.at[r], acc.at[dbuf.at[par, j]], add=True)
        return 0

    lax.fori_loop(0, nblk, block, 0)


def _agg_scratch(NP, D):
    return [
        pltpu.VMEM((2, _BLK, _CHUNK), jnp.int32),
        pltpu.VMEM((2, _BLK, _CHUNK), jnp.int32),
        pltpu.VMEM((_RING, _CHUNK, D), jnp.float32),
        pltpu.VMEM((_ZR, D), jnp.float32),
        pltpu.VMEM_SHARED((NP, D), jnp.float32),
    ] + [pltpu.SemaphoreType.DMA] * _RING


def _sc_agg(src_p, dst_p, hs0, hs1, NP, Dh):
    NR = src_p.shape[0]
    rows_w = NR // _NS
    nblk = rows_w // _BLK
    rows_per = NP // _NS
    mesh = plsc.VectorSubcoreMesh(core_axis_name="c", subcore_axis_name="s")

    @functools.partial(
        pl.kernel,
        out_type=(
            jax.ShapeDtypeStruct((NP, Dh), jnp.float32),
            jax.ShapeDtypeStruct((NP, Dh), jnp.float32),
        ),
        mesh=mesh,
        scratch_types=_agg_scratch(NP, Dh),
    )
    def k(src_h, dst_h, hs0_h, hs1_h, out0_h, out1_h,
          sbuf, dbuf, rows, zbuf, acc, *sems):
        c = lax.axis_index("c")
        s = lax.axis_index("s")
        _zero_acc(zbuf, acc, s, rows_per, Dh)
        base_row = s * rows_w

        def run(hs_h, out_h):
            _agg_pipeline(src_h, dst_h, hs_h, acc, sbuf, dbuf, rows,
                          sems, base_row, nblk)
            plsc.subcore_barrier()
            pltpu.sync_copy(acc.at[pl.ds(s * rows_per, rows_per)],
                            out_h.at[pl.ds(s * rows_per, rows_per)])

        @pl.when(c == 0)
        def _():
            run(hs0_h, out0_h)

        @pl.when(c == 1)
        def _():
            run(hs1_h, out1_h)

    return k(src_p, dst_p, hs0, hs1)


def _sc_agg_esplit(src_p, dst_p, hs, NP, D, blk0=None):
    NR = src_p.shape[0]
    rows_w = NR // (2 * _NS)
    nblk = rows_w // _BLK
    nblk0 = nblk if blk0 is None else blk0
    nblk1 = 2 * nblk - nblk0
    rows_per = NP // _NS
    mesh = plsc.VectorSubcoreMesh(core_axis_name="c", subcore_axis_name="s")

    @functools.partial(
        pl.kernel,
        out_type=(
            jax.ShapeDtypeStruct((NP, D), jnp.float32),
            jax.ShapeDtypeStruct((NP, D), jnp.float32),
        ),
        mesh=mesh,
        scratch_types=_agg_scratch(NP, D),
    )
    def k(src_h, dst_h, hs_h, out0_h, out1_h,
          sbuf, dbuf, rows, zbuf, acc, *sems):
        c = lax.axis_index("c")
        s = lax.axis_index("s")
        _zero_acc(zbuf, acc, s, rows_per, D)
        rw0 = nblk0 * _BLK
        rw1 = nblk1 * _BLK
        base_row = jnp.where(c == 0, s * rw0, _NS * rw0 + s * rw1)
        my_nblk = jnp.where(c == 0, nblk0, nblk1)

        @pl.when(my_nblk > 0)
        def _():
            _agg_pipeline(src_h, dst_h, hs_h, acc, sbuf, dbuf, rows,
                          sems, base_row, my_nblk)

        plsc.subcore_barrier()

        def out(out_h):
            pltpu.sync_copy(acc.at[pl.ds(s * rows_per, rows_per)],
                            out_h.at[pl.ds(s * rows_per, rows_per)])

        @pl.when(c == 0)
        def _():
            out(out0_h)

        @pl.when(c == 1)
        def _():
            out(out1_h)

    return k(src_p, dst_p, hs)


def _tc_pre(xp, W1, hist, bm=1024):
    NP, Din = xp.shape
    Dh = W1.shape[1] // 2
    NW = hist.shape[0]

    def body(x_ref, w_ref, h_ref, hs0_ref, hs1_ref, dis_ref):
        deg = jnp.sum(h_ref[...], axis=0) + 1.0
        dis = lax.rsqrt(deg)
        xs = x_ref[...] * dis[:, None]
        hs = jnp.dot(xs, w_ref[...], preferred_element_type=jnp.float32)
        hs0_ref[...] = hs[:, :Dh]
        hs1_ref[...] = hs[:, Dh:]
        dis_ref[...] = dis[:, None]

    return pl.pallas_call(
        body,
        grid=(NP // bm,),
        in_specs=[
            pl.BlockSpec((bm, Din), lambda i: (i, 0)),
            pl.BlockSpec((Din, 2 * Dh), lambda i: (0, 0)),
            pl.BlockSpec((NW, bm), lambda i: (0, i)),
        ],
        out_specs=[
            pl.BlockSpec((bm, Dh), lambda i: (i, 0)),
            pl.BlockSpec((bm, Dh), lambda i: (i, 0)),
            pl.BlockSpec((bm, 1), lambda i: (i, 0)),
        ],
        out_shape=[
            jax.ShapeDtypeStruct((NP, Dh), jnp.float32),
            jax.ShapeDtypeStruct((NP, Dh), jnp.float32),
            jax.ShapeDtypeStruct((NP, 1), jnp.float32),
        ],
    )(xp, W1, hist)


def _tc_mid(agg0, agg1, hs0, hs1, dis, b1, W2, bm=1024):
    NP, Dh = agg0.shape
    Dout = W2.shape[1]
    Dq = Dout // 2

    def body(a0, a1, h0, h1, d_ref, b_ref, w_ref, o_ref):
        d = d_ref[...]
        b = b_ref[...]
        w = w_ref[...]
        t0 = jnp.maximum(d * (a0[...] + h0[...]) + b[0:1, :Dh], 0.0)
        t1 = jnp.maximum(d * (a1[...] + h1[...]) + b[0:1, Dh:], 0.0)
        o_ref[...] = (jnp.dot(d * t0, w[:Dh], preferred_element_type=jnp.float32)
                      + jnp.dot(d * t1, w[Dh:], preferred_element_type=jnp.float32))

    rb = pl.BlockSpec((bm, Dh), lambda i: (i, 0))
    return pl.pallas_call(
        body,
        grid=(NP // bm,),
        in_specs=[
            rb, rb, rb, rb,
            pl.BlockSpec((bm, 1), lambda i: (i, 0)),
            pl.BlockSpec((8, 2 * Dh), lambda i: (0, 0)),
            pl.BlockSpec((2 * Dh, Dout), lambda i: (0, 0)),
        ],
        out_specs=pl.BlockSpec((bm, Dout), lambda i: (i, 0)),
        out_shape=jax.ShapeDtypeStruct((NP, Dout), jnp.float32),
    )(agg0, agg1, hs0, hs1, dis, b1, W2)


def _tc_post(p0, p1, hs2, dis, b2, bm=1024):
    NP, D = hs2.shape

    def body(a0, a1, h_ref, d_ref, b_ref, o_ref):
        d = d_ref[...]
        b = b_ref[...]
        o_ref[...] = jnp.maximum(
            d * (a0[...] + a1[...] + h_ref[...]) + b[0:1, :], 0.0)

    rb = pl.BlockSpec((bm, D), lambda i: (i, 0))
    return pl.pallas_call(
        body,
        grid=(NP // bm,),
        in_specs=[
            rb, rb, rb,
            pl.BlockSpec((bm, 1), lambda i: (i, 0)),
            pl.BlockSpec((8, D), lambda i: (0, 0)),
        ],
        out_specs=pl.BlockSpec((bm, D), lambda i: (i, 0)),
        out_shape=jax.ShapeDtypeStruct((NP, D), jnp.float32),
    )(p0, p1, hs2, dis, b2)


@jax.jit
def _run(x, edge_index, W1, b1, W2, b2):
    N, Din = x.shape
    E = edge_index.shape[1]
    NP = ((N + 2047) // 2048) * 2048
    blk_edges = 2 * _NS * _BLK * _CHUNK
    EP = ((E + blk_edges - 1) // blk_edges) * blk_edges

    src = edge_index[0].astype(jnp.int32)
    dst = edge_index[1].astype(jnp.int32)
    pad = jnp.full((EP - E,), N, dtype=jnp.int32)
    src_p = jnp.concatenate([src, pad])
    dst_p = jnp.concatenate([dst, pad])
    src_2d = src_p.reshape(EP // _CHUNK, _CHUNK)
    dst_2d = dst_p.reshape(EP // _CHUNK, _CHUNK)
    xp = jnp.concatenate([x, jnp.zeros((NP - N, Din), x.dtype)])
    b1r = jnp.broadcast_to(b1[None, :], (8, b1.shape[0]))
    b2r = jnp.broadcast_to(b2[None, :], (8, b2.shape[0]))

    hist = _sc_deg(dst_p, NP)
    hs0, hs1, dis = _tc_pre(xp, W1, hist)
    agg0, agg1 = _sc_agg(src_2d, dst_2d, hs0, hs1, NP, W1.shape[1] // 2)
    hs2 = _tc_mid(agg0, agg1, hs0, hs1, dis, b1r, W2)
    p0, p1 = _sc_agg_esplit(src_2d, dst_2d, hs2, NP, W2.shape[1], blk0=38)
    y = _tc_post(p0, p1, hs2, dis, b2r)
    return y[:N]


def kernel(x, edge_index, W1, b1, W2, b2):
    return _run(x, edge_index, W1, b1, W2, b2)
